# baseline (speedup 1.0000x reference)
import functools

import jax
import jax.numpy as jnp
from jax import lax
from jax.experimental import pallas as pl
from jax.experimental.pallas import tpu as pltpu
from jax.experimental.pallas import tpu_sc as plsc

_N = 10000
_NPAD = 10240
_E = 320000
_NNEI = 32.0

_NC = 2
_NS = 16
_NW = _NC * _NS
_CB = 80
_CHUNKS_PER_W = _E // (_CB * _NW)
_ROUNDS = 5
_CPR = _CHUNKS_PER_W // _ROUNDS
_ROWS_PER_TILE = _NPAD // _NS


@functools.cache
def _make_agg(D):
  mesh = plsc.VectorSubcoreMesh(core_axis_name="c", subcore_axis_name="s")

  @functools.partial(
      pl.kernel,
      out_type=jax.ShapeDtypeStruct((_NC * _NPAD, D), jnp.float32),
      mesh=mesh,
      scratch_types=[
          pltpu.VMEM((_CPR, _CB), jnp.int32),
          pltpu.VMEM((_CPR, _CB), jnp.int32),
          pltpu.VMEM((_CPR, _CB), jnp.float32),
          pltpu.VMEM((_CB, D), jnp.float32),
          pltpu.VMEM((_CB, D), jnp.float32),
          pltpu.VMEM((_CB, D), jnp.float32),
          pltpu.VMEM_SHARED((_NPAD, D), jnp.float32),
          pltpu.SemaphoreType.DMA,
          pltpu.SemaphoreType.DMA,
          pltpu.SemaphoreType.DMA,
          pltpu.SemaphoreType.DMA,
          pltpu.SemaphoreType.DMA,
          pltpu.SemaphoreType.DMA,
      ],
  )
  def agg(t_hbm, src_hbm, dst_hbm, ew_hbm, zero_hbm, out_hbm,
          src_v, dst_v, ew_v, rows_a, rows_b, rows_c, acc_sh,
          g_a, g_b, g_c, s_a, s_b, s_c):
    core = lax.axis_index("c")
    sub = lax.axis_index("s")
    wid = core * _NS + sub

    r0 = sub * _ROWS_PER_TILE
    pltpu.sync_copy(zero_hbm, acc_sh.at[pl.ds(r0, _ROWS_PER_TILE)])
    plsc.subcore_barrier()

    def start_gather(c, rows, sem):
      pltpu.async_copy(t_hbm.at[src_v.at[c]], rows, sem)

    def wait_gather(c, rows, sem):
      pltpu.make_async_copy(t_hbm.at[src_v.at[c]], rows, sem).wait()

    def start_scat(c, rows, sem):
      pltpu.async_copy(rows, acc_sh.at[dst_v.at[c]], sem, add=True)

    def wait_scat(rows, sem):
      pltpu.make_async_copy(rows, acc_sh.at[pl.ds(0, _CB)], sem).wait()

    def scale(c, rows):
      @pl.loop(0, _CB // 16)
      def _grp(g):
        e0 = g * 16
        w16 = ew_v[c, pl.ds(e0, 16)]
        for i in range(16):
          ws = w16.at[jnp.full((16,), i, jnp.int32)].get(
              mode="promise_in_bounds")
          for k in range(D // 16):
            rows[e0 + i, pl.ds(k * 16, 16)] = (
                rows[e0 + i, pl.ds(k * 16, 16)] * ws)

    for b in range(_ROUNDS):
      pltpu.sync_copy(src_hbm.at[wid, b], src_v)
      pltpu.sync_copy(dst_hbm.at[wid, b], dst_v)
      pltpu.sync_copy(ew_hbm.at[wid, b], ew_v)

      start_gather(0, rows_a, g_a)
      start_gather(1, rows_b, g_b)
      start_gather(2, rows_c, g_c)

      @pl.loop(0, (_CPR - 4) // 3)
      def _trip(t):
        c0 = 3 * t
        wait_gather(c0, rows_a, g_a)
        scale(c0, rows_a)
        start_scat(c0, rows_a, s_a)
        wait_gather(c0 + 1, rows_b, g_b)
        scale(c0 + 1, rows_b)
        start_scat(c0 + 1, rows_b, s_b)
        wait_gather(c0 + 2, rows_c, g_c)
        scale(c0 + 2, rows_c)
        start_scat(c0 + 2, rows_c, s_c)
        wait_scat(rows_a, s_a)
        start_gather(c0 + 3, rows_a, g_a)
        wait_scat(rows_b, s_b)
        start_gather(c0 + 4, rows_b, g_b)
        wait_scat(rows_c, s_c)
        start_gather(c0 + 5, rows_c, g_c)

      e0 = _CPR - 4
      wait_gather(e0, rows_a, g_a)
      scale(e0, rows_a)
      start_scat(e0, rows_a, s_a)
      wait_gather(e0 + 1, rows_b, g_b)
      scale(e0 + 1, rows_b)
      start_scat(e0 + 1, rows_b, s_b)
      wait_scat(rows_a, s_a)
      start_gather(e0 + 3, rows_a, g_a)
      wait_gather(e0 + 2, rows_c, g_c)
      scale(e0 + 2, rows_c)
      start_scat(e0 + 2, rows_c, s_c)
      wait_gather(e0 + 3, rows_a, g_a)
      scale(e0 + 3, rows_a)
      start_scat(e0 + 3, rows_a, s_a)
      wait_scat(rows_a, s_a)
      wait_scat(rows_b, s_b)
      wait_scat(rows_c, s_c)

    plsc.subcore_barrier()

    pltpu.sync_copy(acc_sh.at[pl.ds(r0, _ROWS_PER_TILE)],
                    out_hbm.at[pl.ds(core * _NPAD + r0, _ROWS_PER_TILE)])

  return agg


_BN = 1000


def _mm(x, W, b):
  K = x.shape[1]
  D = W.shape[1]

  def body(h_ref, w_ref, b_ref, o_ref):
    o_ref[...] = jnp.dot(h_ref[...], w_ref[...],
                         preferred_element_type=jnp.float32) + b_ref[...]

  return pl.pallas_call(
      body,
      grid=(_N // _BN,),
      in_specs=[
          pl.BlockSpec((_BN, K), lambda i: (i, 0)),
          pl.BlockSpec((K, D), lambda i: (0, 0)),
          pl.BlockSpec((1, D), lambda i: (0, 0)),
      ],
      out_specs=pl.BlockSpec((_BN, D), lambda i: (i, 0)),
      out_shape=jax.ShapeDtypeStruct((_N, D), jnp.float32),
  )(x, W, b.reshape(1, D))


def _comb_mm(p, W, b):
  K = p.shape[2]
  D = W.shape[1]

  def body(p_ref, w_ref, b_ref, o_ref):
    h = jnp.maximum((p_ref[0] + p_ref[1]) * (1.0 / _NNEI), 0.0)
    o_ref[...] = jnp.dot(h, w_ref[...],
                         preferred_element_type=jnp.float32) + b_ref[...]

  return pl.pallas_call(
      body,
      grid=(_N // _BN,),
      in_specs=[
          pl.BlockSpec((2, _BN, K), lambda i: (0, i, 0)),
          pl.BlockSpec((K, D), lambda i: (0, 0)),
          pl.BlockSpec((1, D), lambda i: (0, 0)),
      ],
      out_specs=pl.BlockSpec((_BN, D), lambda i: (i, 0)),
      out_shape=jax.ShapeDtypeStruct((_N, D), jnp.float32),
  )(p, W, b.reshape(1, D))


def _final_ls(p, D):
  K = p.shape[2]

  def body(p_ref, o_ref):
    h = (p_ref[0, :, :D] + p_ref[1, :, :D]) * (1.0 / _NNEI)
    m = jnp.max(h, axis=1, keepdims=True)
    ex = jnp.exp(h - m)
    s = jnp.log(jnp.sum(ex, axis=1, keepdims=True))
    o_ref[...] = h - m - s

  return pl.pallas_call(
      body,
      grid=(_N // _BN,),
      in_specs=[pl.BlockSpec((2, _BN, K), lambda i: (0, i, 0))],
      out_specs=pl.BlockSpec((_BN, D), lambda i: (i, 0)),
      out_shape=jax.ShapeDtypeStruct((_N, D), jnp.float32),
  )(p)


def kernel(x, edge_index, edge_weight, W0, b0, W1, b1, W2, b2):
  src = edge_index[0].astype(jnp.int32).reshape(_NW, _ROUNDS, _CPR, _CB)
  dst = edge_index[1].astype(jnp.int32).reshape(_NW, _ROUNDS, _CPR, _CB)
  ew = edge_weight.reshape(_NW, _ROUNDS, _CPR, _CB)
  z128 = jnp.zeros((_ROWS_PER_TILE, 128), jnp.float32)
  W2p = jnp.zeros((128, 128), jnp.float32).at[:, :64].set(W2)
  b2p = jnp.zeros((128,), jnp.float32).at[:64].set(b2)

  agg128 = _make_agg(128)
  s = _mm(x, W0, b0)
  p = agg128(s, src, dst, ew, z128).reshape(_NC, _NPAD, 128)
  s = _comb_mm(p, W0, b0)
  p = agg128(s, src, dst, ew, z128).reshape(_NC, _NPAD, 128)
  s = _comb_mm(p, W2p, b2p)
  p = agg128(s, src, dst, ew, z128).reshape(_NC, _NPAD, 128)
  return _final_ls(p, 64)


# speedup vs baseline: 1.0346x; 1.0346x over previous
import functools

import jax
import jax.numpy as jnp
from jax import lax
from jax.experimental import pallas as pl
from jax.experimental.pallas import tpu as pltpu
from jax.experimental.pallas import tpu_sc as plsc

_N = 10000
_NPAD = 10112
_E = 320000
_NNEI = 32.0

_NC = 2
_NS = 16
_NW = _NC * _NS
_CB = 80
_CHUNKS_PER_W = _E // (_CB * _NW)
_ROUNDS = 5
_CPR = _CHUNKS_PER_W // _ROUNDS
_ROWS_PER_TILE = _NPAD // _NS


@functools.cache
def _make_agg(D):
  mesh = plsc.VectorSubcoreMesh(core_axis_name="c", subcore_axis_name="s")

  @functools.partial(
      pl.kernel,
      out_type=jax.ShapeDtypeStruct((_NC * _NPAD, D), jnp.float32),
      mesh=mesh,
      scratch_types=[
          pltpu.VMEM((_CPR * _CB,), jnp.int32),
          pltpu.VMEM((_CPR, _CB), jnp.int32),
          pltpu.VMEM((_CPR * _CB,), jnp.float32),
          pltpu.VMEM((_CB, D), jnp.float32),
          pltpu.VMEM((_CB, D), jnp.float32),
          pltpu.VMEM((_CB, D), jnp.float32),
          pltpu.VMEM((_CB, D), jnp.float32),
          pltpu.VMEM_SHARED((_NPAD, D), jnp.float32),
          pltpu.SemaphoreType.DMA,
          pltpu.SemaphoreType.DMA,
          pltpu.SemaphoreType.DMA,
          pltpu.SemaphoreType.DMA,
          pltpu.SemaphoreType.DMA,
          pltpu.SemaphoreType.DMA,
          pltpu.SemaphoreType.DMA,
          pltpu.SemaphoreType.DMA,
      ],
  )
  def agg(t_hbm, src_hbm, dst_hbm, ew_hbm, zero_hbm, out_hbm,
          src_v, dst_v, ew_v, rows_a, rows_b, rows_c, rows_d, acc_sh,
          g_a, g_b, g_c, g_d, s_a, s_b, s_c, s_d):
    core = lax.axis_index("c")
    sub = lax.axis_index("s")
    wid = core * _NS + sub

    r0 = sub * _ROWS_PER_TILE
    pltpu.sync_copy(zero_hbm, acc_sh.at[pl.ds(r0, _ROWS_PER_TILE)])
    plsc.subcore_barrier()

    def start_gather(c, rows, sem):
      pltpu.async_copy(t_hbm.at[src_v.at[pl.ds(c * _CB, _CB)]], rows, sem)

    def wait_gather(c, rows, sem):
      pltpu.make_async_copy(
          t_hbm.at[src_v.at[pl.ds(c * _CB, _CB)]], rows, sem).wait()

    def start_scat(c, rows, sem):
      pltpu.async_copy(rows, acc_sh.at[dst_v.at[c]], sem, add=True)

    def wait_scat(rows, sem):
      pltpu.make_async_copy(rows, acc_sh.at[pl.ds(0, _CB)], sem).wait()

    def scale(c, rows):
      @pl.loop(0, _CB // 16)
      def _grp(g):
        e0 = g * 16
        w16 = ew_v[pl.ds(c * _CB + e0, 16)]
        for i in range(16):
          ws = w16.at[jnp.full((16,), i, jnp.int32)].get(
              mode="promise_in_bounds")
          for k in range(D // 16):
            rows[e0 + i, pl.ds(k * 16, 16)] = (
                rows[e0 + i, pl.ds(k * 16, 16)] * ws)

    for b in range(_ROUNDS):
      rbase = (wid * _ROUNDS + b) * _CPR * _CB
      pltpu.sync_copy(src_hbm.at[pl.ds(rbase, _CPR * _CB)], src_v)
      pltpu.sync_copy(dst_hbm.at[wid, b], dst_v)
      pltpu.sync_copy(ew_hbm.at[pl.ds(rbase, _CPR * _CB)], ew_v)

      start_gather(0, rows_a, g_a)
      start_gather(1, rows_b, g_b)
      start_gather(2, rows_c, g_c)
      start_gather(3, rows_d, g_d)

      @pl.loop(0, _CPR // 4)
      def _quad(t):
        c0 = 4 * t
        for j, (rows, gs, ss) in enumerate(
            [(rows_a, g_a, s_a), (rows_b, g_b, s_b),
             (rows_c, g_c, s_c), (rows_d, g_d, s_d)]):
          wait_gather(c0 + j, rows, gs)
          scale(c0 + j, rows)
          start_scat(c0 + j, rows, ss)
        for j, (rows, gs, ss) in enumerate(
            [(rows_a, g_a, s_a), (rows_b, g_b, s_b),
             (rows_c, g_c, s_c), (rows_d, g_d, s_d)]):
          wait_scat(rows, ss)

          @pl.when(c0 + 4 + j < _CPR)
          def _():
            start_gather(c0 + 4 + j, rows, gs)

      wait_gather(_CPR - 1, rows_a, g_a)
      scale(_CPR - 1, rows_a)
      start_scat(_CPR - 1, rows_a, s_a)
      wait_scat(rows_a, s_a)

    plsc.subcore_barrier()

    pltpu.sync_copy(acc_sh.at[pl.ds(r0, _ROWS_PER_TILE)],
                    out_hbm.at[pl.ds(core * _NPAD + r0, _ROWS_PER_TILE)])

  return agg


_BN = 1000


def _mm(x, W, b):
  K = x.shape[1]
  D = W.shape[1]

  def body(h_ref, w_ref, b_ref, o_ref):
    o_ref[...] = jnp.dot(h_ref[...], w_ref[...],
                         preferred_element_type=jnp.float32) + b_ref[...]

  return pl.pallas_call(
      body,
      grid=(_N // _BN,),
      in_specs=[
          pl.BlockSpec((_BN, K), lambda i: (i, 0)),
          pl.BlockSpec((K, D), lambda i: (0, 0)),
          pl.BlockSpec((1, D), lambda i: (0, 0)),
      ],
      out_specs=pl.BlockSpec((_BN, D), lambda i: (i, 0)),
      out_shape=jax.ShapeDtypeStruct((_N, D), jnp.float32),
  )(x, W, b.reshape(1, D))


def _comb_mm(p, W, b):
  K = p.shape[2]
  D = W.shape[1]

  def body(p_ref, w_ref, b_ref, o_ref):
    h = jnp.maximum((p_ref[0] + p_ref[1]) * (1.0 / _NNEI), 0.0)
    o_ref[...] = jnp.dot(h, w_ref[...],
                         preferred_element_type=jnp.float32) + b_ref[...]

  return pl.pallas_call(
      body,
      grid=(_N // _BN,),
      in_specs=[
          pl.BlockSpec((2, _BN, K), lambda i: (0, i, 0)),
          pl.BlockSpec((K, D), lambda i: (0, 0)),
          pl.BlockSpec((1, D), lambda i: (0, 0)),
      ],
      out_specs=pl.BlockSpec((_BN, D), lambda i: (i, 0)),
      out_shape=jax.ShapeDtypeStruct((_N, D), jnp.float32),
  )(p, W, b.reshape(1, D))


def _final_ls(p, D):
  K = p.shape[2]

  def body(p_ref, o_ref):
    h = (p_ref[0, :, :D] + p_ref[1, :, :D]) * (1.0 / _NNEI)
    m = jnp.max(h, axis=1, keepdims=True)
    ex = jnp.exp(h - m)
    s = jnp.log(jnp.sum(ex, axis=1, keepdims=True))
    o_ref[...] = h - m - s

  return pl.pallas_call(
      body,
      grid=(_N // _BN,),
      in_specs=[pl.BlockSpec((2, _BN, K), lambda i: (0, i, 0))],
      out_specs=pl.BlockSpec((_BN, D), lambda i: (i, 0)),
      out_shape=jax.ShapeDtypeStruct((_N, D), jnp.float32),
  )(p)


def kernel(x, edge_index, edge_weight, W0, b0, W1, b1, W2, b2):
  src = edge_index[0].astype(jnp.int32)
  dst = edge_index[1].astype(jnp.int32).reshape(_NW, _ROUNDS, _CPR, _CB)
  ew = edge_weight
  z128 = jnp.zeros((_ROWS_PER_TILE, 128), jnp.float32)
  W2p = jnp.zeros((128, 128), jnp.float32).at[:, :64].set(W2)
  b2p = jnp.zeros((128,), jnp.float32).at[:64].set(b2)

  agg128 = _make_agg(128)
  s = _mm(x, W0, b0)
  p = agg128(s, src, dst, ew, z128).reshape(_NC, _NPAD, 128)
  s = _comb_mm(p, W0, b0)
  p = agg128(s, src, dst, ew, z128).reshape(_NC, _NPAD, 128)
  s = _comb_mm(p, W2p, b2p)
  p = agg128(s, src, dst, ew, z128).reshape(_NC, _NPAD, 128)
  return _final_ls(p, 64)


# speedup vs baseline: 1.1205x; 1.0830x over previous
import functools

import jax
import jax.numpy as jnp
from jax import lax
from jax.experimental import pallas as pl
from jax.experimental.pallas import tpu as pltpu
from jax.experimental.pallas import tpu_sc as plsc

_N = 10000
_NPAD = 10112
_E = 320000
_NNEI = 32.0

_NC = 2
_NS = 16
_NW = _NC * _NS
_CB = 80
_CHUNKS_PER_W = _E // (_CB * _NW)
_ROUNDS = 5
_CPR = _CHUNKS_PER_W // _ROUNDS
_ROWS_PER_TILE = _NPAD // _NS


@functools.cache
def _make_agg(D):
  mesh = plsc.VectorSubcoreMesh(core_axis_name="c", subcore_axis_name="s")

  @functools.partial(
      pl.kernel,
      out_type=jax.ShapeDtypeStruct((_NC * _NPAD, D), jnp.float32),
      mesh=mesh,
      scratch_types=[
          pltpu.VMEM((_CPR * _CB,), jnp.int32),
          pltpu.VMEM((_CPR, _CB), jnp.int32),
          pltpu.VMEM((_CPR * _CB,), jnp.float32),
          pltpu.VMEM((_CB, D), jnp.float32),
          pltpu.VMEM((_CB, D), jnp.float32),
          pltpu.VMEM((_CB, D), jnp.float32),
          pltpu.VMEM((_CB, D), jnp.float32),
          pltpu.VMEM_SHARED((_NPAD, D), jnp.float32),
          pltpu.SemaphoreType.DMA,
          pltpu.SemaphoreType.DMA,
          pltpu.SemaphoreType.DMA,
          pltpu.SemaphoreType.DMA,
          pltpu.SemaphoreType.DMA,
          pltpu.SemaphoreType.DMA,
          pltpu.SemaphoreType.DMA,
          pltpu.SemaphoreType.DMA,
      ],
  )
  def agg(t_hbm, src_hbm, dst_hbm, ew_hbm, zero_hbm, out_hbm,
          src_v, dst_v, ew_v, rows_a, rows_b, rows_c, rows_d, acc_sh,
          g_a, g_b, g_c, g_d, s_a, s_b, s_c, s_d):
    core = lax.axis_index("c")
    sub = lax.axis_index("s")
    wid = core * _NS + sub

    r0 = sub * _ROWS_PER_TILE
    pltpu.sync_copy(zero_hbm, acc_sh.at[pl.ds(r0, _ROWS_PER_TILE)])
    plsc.subcore_barrier()

    def start_gather(c, rows, sem):
      pltpu.async_copy(t_hbm.at[src_v.at[pl.ds(c * _CB, _CB)]], rows, sem)

    def wait_gather(c, rows, sem):
      pltpu.make_async_copy(
          t_hbm.at[src_v.at[pl.ds(c * _CB, _CB)]], rows, sem).wait()

    def start_scat(c, rows, sem):
      pltpu.async_copy(rows, acc_sh.at[dst_v.at[c]], sem, add=True)

    def wait_scat(rows, sem):
      pltpu.make_async_copy(rows, acc_sh.at[pl.ds(0, _CB)], sem).wait()

    def scale(c, rows):
      @pl.loop(0, _CB // 16)
      def _grp(g):
        e0 = g * 16
        w16 = ew_v[pl.ds(c * _CB + e0, 16)]
        for i in range(16):
          ws = w16.at[jnp.full((16,), i, jnp.int32)].get(
              mode="promise_in_bounds")
          for k in range(D // 16):
            rows[e0 + i, pl.ds(k * 16, 16)] = (
                rows[e0 + i, pl.ds(k * 16, 16)] * ws)

    for b in range(_ROUNDS):
      rbase = (wid * _ROUNDS + b) * _CPR * _CB
      pltpu.sync_copy(src_hbm.at[pl.ds(rbase, _CPR * _CB)], src_v)
      pltpu.sync_copy(dst_hbm.at[wid, b], dst_v)
      pltpu.sync_copy(ew_hbm.at[pl.ds(rbase, _CPR * _CB)], ew_v)

      start_gather(0, rows_a, g_a)
      start_gather(1, rows_b, g_b)
      start_gather(2, rows_c, g_c)
      start_gather(3, rows_d, g_d)

      ring = [(rows_a, g_a, s_a), (rows_b, g_b, s_b),
              (rows_c, g_c, s_c), (rows_d, g_d, s_d)]

      @pl.loop(0, _CPR // 4)
      def _quad(t):
        c0 = 4 * t
        for j, (rows, gs, ss) in enumerate(ring):
          wait_gather(c0 + j, rows, gs)
          scale(c0 + j, rows)
          start_scat(c0 + j, rows, ss)
          prows, pgs, pss = ring[(j + 3) % 4]
          if j >= 1:
            wait_scat(prows, pss)

            @pl.when(c0 + 3 + j < _CPR)
            def _():
              start_gather(c0 + 3 + j, prows, pgs)
        rows, gs, ss = ring[3]
        wait_scat(rows, ss)

        @pl.when(c0 + 7 < _CPR)
        def _():
          start_gather(c0 + 7, rows, gs)

      wait_gather(_CPR - 1, rows_a, g_a)
      scale(_CPR - 1, rows_a)
      start_scat(_CPR - 1, rows_a, s_a)
      wait_scat(rows_a, s_a)

    plsc.subcore_barrier()

    pltpu.sync_copy(acc_sh.at[pl.ds(r0, _ROWS_PER_TILE)],
                    out_hbm.at[pl.ds(core * _NPAD + r0, _ROWS_PER_TILE)])

  return agg


_BN = 1000


def _mm(x, W, b):
  K = x.shape[1]
  D = W.shape[1]

  def body(h_ref, w_ref, b_ref, o_ref):
    o_ref[...] = jnp.dot(h_ref[...], w_ref[...],
                         preferred_element_type=jnp.float32) + b_ref[...]

  return pl.pallas_call(
      body,
      grid=(_N // _BN,),
      in_specs=[
          pl.BlockSpec((_BN, K), lambda i: (i, 0)),
          pl.BlockSpec((K, D), lambda i: (0, 0)),
          pl.BlockSpec((1, D), lambda i: (0, 0)),
      ],
      out_specs=pl.BlockSpec((_BN, D), lambda i: (i, 0)),
      out_shape=jax.ShapeDtypeStruct((_N, D), jnp.float32),
  )(x, W, b.reshape(1, D))


def _comb_mm(p, W, b):
  K = p.shape[2]
  D = W.shape[1]

  def body(p_ref, w_ref, b_ref, o_ref):
    h = jnp.maximum((p_ref[0] + p_ref[1]) * (1.0 / _NNEI), 0.0)
    o_ref[...] = jnp.dot(h, w_ref[...],
                         preferred_element_type=jnp.float32) + b_ref[...]

  return pl.pallas_call(
      body,
      grid=(_N // _BN,),
      in_specs=[
          pl.BlockSpec((2, _BN, K), lambda i: (0, i, 0)),
          pl.BlockSpec((K, D), lambda i: (0, 0)),
          pl.BlockSpec((1, D), lambda i: (0, 0)),
      ],
      out_specs=pl.BlockSpec((_BN, D), lambda i: (i, 0)),
      out_shape=jax.ShapeDtypeStruct((_N, D), jnp.float32),
  )(p, W, b.reshape(1, D))


def _final_ls(p, D):
  K = p.shape[2]

  def body(p_ref, o_ref):
    h = (p_ref[0, :, :D] + p_ref[1, :, :D]) * (1.0 / _NNEI)
    m = jnp.max(h, axis=1, keepdims=True)
    ex = jnp.exp(h - m)
    s = jnp.log(jnp.sum(ex, axis=1, keepdims=True))
    o_ref[...] = h - m - s

  return pl.pallas_call(
      body,
      grid=(_N // _BN,),
      in_specs=[pl.BlockSpec((2, _BN, K), lambda i: (0, i, 0))],
      out_specs=pl.BlockSpec((_BN, D), lambda i: (i, 0)),
      out_shape=jax.ShapeDtypeStruct((_N, D), jnp.float32),
  )(p)


def kernel(x, edge_index, edge_weight, W0, b0, W1, b1, W2, b2):
  src = edge_index[0].astype(jnp.int32)
  dst = edge_index[1].astype(jnp.int32).reshape(_NW, _ROUNDS, _CPR, _CB)
  ew = edge_weight
  z128 = jnp.zeros((_ROWS_PER_TILE, 128), jnp.float32)
  W2p = jnp.zeros((128, 128), jnp.float32).at[:, :64].set(W2)
  b2p = jnp.zeros((128,), jnp.float32).at[:64].set(b2)

  agg128 = _make_agg(128)
  s = _mm(x, W0, b0)
  p = agg128(s, src, dst, ew, z128).reshape(_NC, _NPAD, 128)
  s = _comb_mm(p, W0, b0)
  p = agg128(s, src, dst, ew, z128).reshape(_NC, _NPAD, 128)
  s = _comb_mm(p, W2p, b2p)
  p = agg128(s, src, dst, ew, z128).reshape(_NC, _NPAD, 128)
  return _final_ls(p, 64)
